# baseline (speedup 1.0000x reference)
import functools

import jax
import jax.numpy as jnp
from jax import lax
from jax.experimental import pallas as pl
from jax.experimental.pallas import tpu as pltpu
from jax.experimental.pallas import tpu_sc as plsc

D = 128
CH = 64
NB = 3
NC = 2
NS = 16
NW = NC * NS


def _acc_rows(n_nodes):
  u = NS * 8
  return ((n_nodes + 1 + u - 1) // u) * u


def _mesh():
  return plsc.VectorSubcoreMesh(
      core_axis_name="c", subcore_axis_name="s", num_cores=NC, num_subcores=NS
  )


def _make_sc_agg(n_nodes, nch):
  nr = _acc_rows(n_nodes)
  rpt = nr // NS
  assert nch % (2 * NB) == 0 and nch >= 2 * NB
  nch2 = nch // 2

  out_type = jax.ShapeDtypeStruct((NC, nr, D), jnp.float32)
  scratch = [
      pltpu.VMEM((nch2, 2 * CH), jnp.int32),
      pltpu.VMEM((nch2, 2 * CH), jnp.int32),
      [pltpu.VMEM((CH, D), jnp.float32) for _ in range(NB)],
      pltpu.MemorySpace.VMEM_SHARED((nr, D), jnp.float32),
      [pltpu.SemaphoreType.DMA for _ in range(NB)],
      [pltpu.SemaphoreType.DMA for _ in range(NB)],
  ]

  def body(y, srcp, dstp, z128_h, psum,
           src_v, dst_v, rows, acc_p, gsem, ssem):
    c = lax.axis_index("c")
    s = lax.axis_index("s")
    w = c * NS + s

    pltpu.sync_copy(srcp.at[w], src_v)
    pltpu.sync_copy(dstp.at[w], dst_v)
    pltpu.sync_copy(z128_h, rows[0])

    nz = rpt // CH
    rem = rpt - nz * CH

    @pl.loop(0, nz)
    def _(k):
      pltpu.sync_copy(rows[0], acc_p.at[pl.ds(s * rpt + k * CH, CH)])

    if rem:
      pltpu.sync_copy(rows[0].at[pl.ds(0, rem)],
                      acc_p.at[pl.ds(s * rpt + nz * CH, rem)])

    plsc.subcore_barrier()

    def idx(v, q):
      return v.at[q // 2, pl.ds((q % 2) * CH, CH)]

    def fire_g(q, b):
      return pltpu.async_copy(y.at[idx(src_v, q)], rows[b], gsem[b])

    def wait_g(q, b):
      pltpu.make_async_copy(y.at[idx(src_v, q)], rows[b], gsem[b]).wait()

    def fire_s(q, b):
      return pltpu.async_copy(rows[b], acc_p.at[idx(dst_v, q)], ssem[b],
                              add=True)

    def wait_s(q, b):
      pltpu.make_async_copy(rows[b], acc_p.at[idx(dst_v, q)], ssem[b]).wait()

    fire_g(0, 0)
    for b in range(2):
      wait_g(b, b)
      fire_s(b, b)
      fire_g(b + 1, (b + 1) % NB)
    wait_g(2, 2 % NB)
    fire_s(2, 2 % NB)
    wait_s(0, 0)
    fire_g(3, 3 % NB)

    @pl.loop(1, nch // NB - 1)
    def _(m):
      qb = m * NB
      for b in range(NB):
        q = qb + b
        wait_g(q, b)
        fire_s(q, b)
        bn = (b + 1) % NB
        wait_s(q - 2, bn)
        fire_g(q + 1, bn)

    qb = nch - NB
    for off in range(NB):
      q = qb + off
      b = q % NB
      wait_g(q, b)
      fire_s(q, b)
      if off < NB - 1:
        bn = (b + 1) % NB
        wait_s(q - 2, bn)
        fire_g(q + 1, bn)
    wait_s(nch - 3, (nch - 3) % NB)
    wait_s(nch - 2, (nch - 2) % NB)
    wait_s(nch - 1, (nch - 1) % NB)

    plsc.subcore_barrier()

    sl = pl.ds(s * rpt, rpt)
    pltpu.sync_copy(acc_p.at[sl], psum.at[c, sl])

  return pl.kernel(body, out_type=out_type, mesh=_mesh(), scratch_types=scratch)


def _make_sc_counts(n_nodes, nch):
  nr = _acc_rows(n_nodes)
  rpt = nr // NS
  nch2 = nch // 2

  out_type = jax.ShapeDtypeStruct((NC, nr, D), jnp.float32)
  scratch = [
      pltpu.VMEM((nch2, 2 * CH), jnp.int32),
      pltpu.VMEM((2 * CH, D), jnp.float32),
      pltpu.VMEM((CH, D), jnp.float32),
      pltpu.MemorySpace.VMEM_SHARED((nr, D), jnp.float32),
      pltpu.SemaphoreType.DMA,
      pltpu.SemaphoreType.DMA,
  ]

  def body(dstp, ones_h, z128_h, cnt,
           dst_v, ones_v, zc_v, acc_c, ssa, ssb):
    c = lax.axis_index("c")
    s = lax.axis_index("s")
    w = c * NS + s

    pltpu.sync_copy(dstp.at[w], dst_v)
    pltpu.sync_copy(z128_h, zc_v)
    pltpu.sync_copy(ones_h, ones_v)

    nz = rpt // CH
    rem = rpt - nz * CH

    @pl.loop(0, nz)
    def _(k):
      pltpu.sync_copy(zc_v, acc_c.at[pl.ds(s * rpt + k * CH, CH)])

    if rem:
      pltpu.sync_copy(zc_v.at[pl.ds(0, rem)],
                      acc_c.at[pl.ds(s * rpt + nz * CH, rem)])

    plsc.subcore_barrier()

    even = nch2 - (nch2 % 2)

    @pl.loop(0, even, step=2)
    def _(m):
      sa = pltpu.async_copy(ones_v, acc_c.at[dst_v.at[m]], ssa, add=True)
      sb = pltpu.async_copy(ones_v, acc_c.at[dst_v.at[m + 1]], ssb, add=True)
      sa.wait()
      sb.wait()

    if nch2 % 2:
      pltpu.async_copy(ones_v, acc_c.at[dst_v.at[nch2 - 1]], ssa,
                       add=True).wait()

    plsc.subcore_barrier()

    sl = pl.ds(s * rpt, rpt)
    pltpu.sync_copy(acc_c.at[sl], cnt.at[c, sl])

  return pl.kernel(body, out_type=out_type, mesh=_mesh(), scratch_types=scratch)


_BR = 1000


def _mm_body(x_ref, w_ref, o_ref):
  o_ref[...] = jnp.dot(
      x_ref[...], w_ref[...],
      preferred_element_type=jnp.float32, precision=lax.Precision.HIGHEST,
  )


def _combine_body(p0, p1, c0, c1, h, wr, b, wl, hn_o, y_o):
  cnt = jnp.maximum(c0[0, :, 0:1] + c1[0, :, 0:1], 1.0)
  z = (p0[0] + p1[0]) / cnt
  z = z + jnp.dot(h[...], wr[...], preferred_element_type=jnp.float32,
                  precision=lax.Precision.HIGHEST)
  z = z + b[...]
  hn = jnp.where(z >= 0, z, 0.1 * z)
  hn_o[...] = hn
  y_o[...] = jnp.dot(hn, wl[...], preferred_element_type=jnp.float32,
                     precision=lax.Precision.HIGHEST)


def _final_body(p0, p1, c0, c1, h, wr, b, o_ref):
  cnt = jnp.maximum(c0[0, :, 0:1] + c1[0, :, 0:1], 1.0)
  z = (p0[0] + p1[0]) / cnt
  z = z + jnp.dot(h[...], wr[...], preferred_element_type=jnp.float32,
                  precision=lax.Precision.HIGHEST)
  z = z + b[...]
  m = jnp.max(z, axis=1, keepdims=True)
  zs = z - m
  o_ref[...] = zs - jnp.log(jnp.sum(jnp.exp(zs), axis=1, keepdims=True))


def _row_spec(width=D):
  return pl.BlockSpec((_BR, width), lambda i: (i, 0))


def _core_spec(core, width=D):
  return pl.BlockSpec((1, _BR, width), lambda i, c=core: (c, i, 0))


def _full_spec(shape):
  return pl.BlockSpec(shape, lambda i: (0,) * len(shape))


def _make_tc_kernels(n):
  grid = (n // _BR,)
  mm = pl.pallas_call(
      _mm_body,
      grid=grid,
      in_specs=[_row_spec(), _full_spec((D, D))],
      out_specs=_row_spec(),
      out_shape=jax.ShapeDtypeStruct((n, D), jnp.float32),
  )
  combine = pl.pallas_call(
      _combine_body,
      grid=grid,
      in_specs=[_core_spec(0), _core_spec(1), _core_spec(0), _core_spec(1),
                _row_spec(), _full_spec((D, D)), _full_spec((1, D)),
                _full_spec((D, D))],
      out_specs=[_row_spec(), _row_spec()],
      out_shape=[jax.ShapeDtypeStruct((n, D), jnp.float32),
                 jax.ShapeDtypeStruct((n, D), jnp.float32)],
  )
  final = pl.pallas_call(
      _final_body,
      grid=grid,
      in_specs=[_core_spec(0), _core_spec(1), _core_spec(0), _core_spec(1),
                _row_spec(), _full_spec((D, D)), _full_spec((1, D))],
      out_specs=_row_spec(),
      out_shape=jax.ShapeDtypeStruct((n, D), jnp.float32),
  )
  return mm, combine, final


def kernel(x, edge_index, enc_Wl, enc_Wr, enc_b, l0_Wl, l0_Wr, l0_b,
           l1_Wl, l1_Wr, l1_b, dec_Wl, dec_Wr, dec_b):
  n, d = x.shape
  assert d == D
  e = edge_index.shape[1]

  nch = -(-e // (NW * CH))
  nch = ((nch + 2 * NB - 1) // (2 * NB)) * (2 * NB)
  pad = NW * nch * CH - e
  src = jnp.concatenate(
      [edge_index[0].astype(jnp.int32), jnp.zeros((pad,), jnp.int32)]
  ).reshape(NW, nch // 2, 2 * CH)
  dst = jnp.concatenate(
      [edge_index[1].astype(jnp.int32), jnp.full((pad,), n, jnp.int32)]
  ).reshape(NW, nch // 2, 2 * CH)

  ones2 = jnp.ones((2 * CH, D), jnp.float32)
  z128 = jnp.zeros((CH, D), jnp.float32)

  sc_counts = _make_sc_counts(n, nch)
  sc_agg = _make_sc_agg(n, nch)
  mm, combine, final = _make_tc_kernels(n)

  b_enc = enc_b.reshape(1, D)
  b_l0 = l0_b.reshape(1, D)
  b_l1 = l1_b.reshape(1, D)
  b_dec = dec_b.reshape(1, D)

  cnt = sc_counts(dst, ones2, z128)
  y = mm(x, enc_Wl)
  p = sc_agg(y, src, dst, z128)
  h, y = combine(p, p, cnt, cnt, x, enc_Wr, b_enc, l0_Wl)
  p = sc_agg(y, src, dst, z128)
  h, y = combine(p, p, cnt, cnt, h, l0_Wr, b_l0, l1_Wl)
  p = sc_agg(y, src, dst, z128)
  h, y = combine(p, p, cnt, cnt, h, l1_Wr, b_l1, dec_Wl)
  p = sc_agg(y, src, dst, z128)
  return final(p, p, cnt, cnt, h, dec_Wr, b_dec)


# speedup vs baseline: 1.3401x; 1.3401x over previous
import functools

import jax
import jax.numpy as jnp
from jax import lax
from jax.experimental import pallas as pl
from jax.experimental.pallas import tpu as pltpu
from jax.experimental.pallas import tpu_sc as plsc

D = 128
CH = 64
NB = 3
NC = 2
NS = 16
NW = NC * NS


def _acc_rows(n_nodes):
  u = NS * 8
  return ((n_nodes + 1 + u - 1) // u) * u


def _mesh():
  return plsc.VectorSubcoreMesh(
      core_axis_name="c", subcore_axis_name="s", num_cores=NC, num_subcores=NS
  )


def _make_sc_agg(n_nodes, nch, mode="both"):
  nr = _acc_rows(n_nodes)
  rpt = nr // NS
  SBD = 16
  assert nch % SBD == 0
  GCH = 2 * CH

  out_type = jax.ShapeDtypeStruct((NC, nr, D), jnp.float32)
  scratch = [
      pltpu.VMEM((nch // 2, GCH), jnp.int32),
      pltpu.VMEM((SBD, GCH), jnp.int32),
      pltpu.VMEM((GCH, D), jnp.float32),
      pltpu.VMEM((GCH, D), jnp.float32),
      pltpu.MemorySpace.VMEM_SHARED((nr, D), jnp.float32),
      pltpu.SemaphoreType.DMA,
      pltpu.SemaphoreType.DMA,
      pltpu.SemaphoreType.DMA,
      pltpu.SemaphoreType.DMA,
  ]

  def body(y, srcp, dstp, z128_h, psum,
           src_v, dst_v, rows_a, rows_b, acc_p, sga, sgb, ssa, ssb):
    c = lax.axis_index("c")
    s = lax.axis_index("s")
    w = c * NS + s

    pltpu.sync_copy(srcp.at[w], src_v)
    pltpu.sync_copy(z128_h, rows_a.at[pl.ds(0, CH)])
    pltpu.sync_copy(z128_h, rows_a.at[pl.ds(CH, CH)])
    pltpu.sync_copy(z128_h, rows_b.at[pl.ds(0, CH)])
    pltpu.sync_copy(z128_h, rows_b.at[pl.ds(CH, CH)])

    nz = rpt // CH
    rem = rpt - nz * CH

    @pl.loop(0, nz)
    def _(k):
      pltpu.sync_copy(rows_a.at[pl.ds(0, CH)],
                      acc_p.at[pl.ds(s * rpt + k * CH, CH)])

    if rem:
      pltpu.sync_copy(rows_a.at[pl.ds(0, rem)],
                      acc_p.at[pl.ds(s * rpt + nz * CH, rem)])

    plsc.subcore_barrier()

    nbh = nch // 2 // SBD

    @pl.loop(0, nbh)
    def _(bo):
      pltpu.sync_copy(dstp.at[w, pl.ds(bo * SBD, SBD)], dst_v)

      @pl.loop(0, SBD, step=2)
      def _(j):
        q = bo * SBD + j
        if mode != "scatter":
          ga = pltpu.async_copy(y.at[src_v.at[q]], rows_a, sga)
          ga.wait()
        if mode != "gather":
          sa = pltpu.async_copy(rows_a, acc_p.at[dst_v.at[j]], ssa, add=True)
        if mode != "scatter":
          gb = pltpu.async_copy(y.at[src_v.at[q + 1]], rows_b, sgb)
          gb.wait()
        if mode != "gather":
          sb = pltpu.async_copy(rows_b, acc_p.at[dst_v.at[j + 1]], ssb,
                                add=True)
          sa.wait()
          sb.wait()

    plsc.subcore_barrier()

    sl = pl.ds(s * rpt, rpt)
    pltpu.sync_copy(acc_p.at[sl], psum.at[c, sl])

  return pl.kernel(body, out_type=out_type, mesh=_mesh(), scratch_types=scratch)


def _make_sc_counts(n_nodes, nch):
  nr = _acc_rows(n_nodes)
  rpt = nr // NS
  nch2 = nch // 2

  out_type = jax.ShapeDtypeStruct((NC, nr, D), jnp.float32)
  scratch = [
      pltpu.VMEM((nch2, 2 * CH), jnp.int32),
      pltpu.VMEM((2 * CH, D), jnp.float32),
      pltpu.VMEM((CH, D), jnp.float32),
      pltpu.MemorySpace.VMEM_SHARED((nr, D), jnp.float32),
      pltpu.SemaphoreType.DMA,
      pltpu.SemaphoreType.DMA,
  ]

  def body(dstp, ones_h, z128_h, cnt,
           dst_v, ones_v, zc_v, acc_c, ssa, ssb):
    c = lax.axis_index("c")
    s = lax.axis_index("s")
    w = c * NS + s

    pltpu.sync_copy(dstp.at[w], dst_v)
    pltpu.sync_copy(z128_h, zc_v)
    pltpu.sync_copy(ones_h, ones_v)

    nz = rpt // CH
    rem = rpt - nz * CH

    @pl.loop(0, nz)
    def _(k):
      pltpu.sync_copy(zc_v, acc_c.at[pl.ds(s * rpt + k * CH, CH)])

    if rem:
      pltpu.sync_copy(zc_v.at[pl.ds(0, rem)],
                      acc_c.at[pl.ds(s * rpt + nz * CH, rem)])

    plsc.subcore_barrier()

    even = nch2 - (nch2 % 2)

    @pl.loop(0, even, step=2)
    def _(m):
      sa = pltpu.async_copy(ones_v, acc_c.at[dst_v.at[m]], ssa, add=True)
      sb = pltpu.async_copy(ones_v, acc_c.at[dst_v.at[m + 1]], ssb, add=True)
      sa.wait()
      sb.wait()

    if nch2 % 2:
      pltpu.async_copy(ones_v, acc_c.at[dst_v.at[nch2 - 1]], ssa,
                       add=True).wait()

    plsc.subcore_barrier()

    sl = pl.ds(s * rpt, rpt)
    pltpu.sync_copy(acc_c.at[sl], cnt.at[c, sl])

  return pl.kernel(body, out_type=out_type, mesh=_mesh(), scratch_types=scratch)


_BR = 1000


def _mm_body(x_ref, w_ref, o_ref):
  o_ref[...] = jnp.dot(
      x_ref[...], w_ref[...],
      preferred_element_type=jnp.float32, precision=lax.Precision.HIGHEST,
  )


def _combine_body(p0, p1, c0, c1, h, wr, b, wl, hn_o, y_o):
  cnt = jnp.maximum(c0[0, :, 0:1] + c1[0, :, 0:1], 1.0)
  z = (p0[0] + p1[0]) / cnt
  z = z + jnp.dot(h[...], wr[...], preferred_element_type=jnp.float32,
                  precision=lax.Precision.HIGHEST)
  z = z + b[...]
  hn = jnp.where(z >= 0, z, 0.1 * z)
  hn_o[...] = hn
  y_o[...] = jnp.dot(hn, wl[...], preferred_element_type=jnp.float32,
                     precision=lax.Precision.HIGHEST)


def _final_body(p0, p1, c0, c1, h, wr, b, o_ref):
  cnt = jnp.maximum(c0[0, :, 0:1] + c1[0, :, 0:1], 1.0)
  z = (p0[0] + p1[0]) / cnt
  z = z + jnp.dot(h[...], wr[...], preferred_element_type=jnp.float32,
                  precision=lax.Precision.HIGHEST)
  z = z + b[...]
  m = jnp.max(z, axis=1, keepdims=True)
  zs = z - m
  o_ref[...] = zs - jnp.log(jnp.sum(jnp.exp(zs), axis=1, keepdims=True))


def _row_spec(width=D):
  return pl.BlockSpec((_BR, width), lambda i: (i, 0))


def _core_spec(core, width=D):
  return pl.BlockSpec((1, _BR, width), lambda i, c=core: (c, i, 0))


def _full_spec(shape):
  return pl.BlockSpec(shape, lambda i: (0,) * len(shape))


def _make_tc_kernels(n):
  grid = (n // _BR,)
  mm = pl.pallas_call(
      _mm_body,
      grid=grid,
      in_specs=[_row_spec(), _full_spec((D, D))],
      out_specs=_row_spec(),
      out_shape=jax.ShapeDtypeStruct((n, D), jnp.float32),
  )
  combine = pl.pallas_call(
      _combine_body,
      grid=grid,
      in_specs=[_core_spec(0), _core_spec(1), _core_spec(0), _core_spec(1),
                _row_spec(), _full_spec((D, D)), _full_spec((1, D)),
                _full_spec((D, D))],
      out_specs=[_row_spec(), _row_spec()],
      out_shape=[jax.ShapeDtypeStruct((n, D), jnp.float32),
                 jax.ShapeDtypeStruct((n, D), jnp.float32)],
  )
  final = pl.pallas_call(
      _final_body,
      grid=grid,
      in_specs=[_core_spec(0), _core_spec(1), _core_spec(0), _core_spec(1),
                _row_spec(), _full_spec((D, D)), _full_spec((1, D))],
      out_specs=_row_spec(),
      out_shape=jax.ShapeDtypeStruct((n, D), jnp.float32),
  )
  return mm, combine, final


def kernel(x, edge_index, enc_Wl, enc_Wr, enc_b, l0_Wl, l0_Wr, l0_b,
           l1_Wl, l1_Wr, l1_b, dec_Wl, dec_Wr, dec_b):
  n, d = x.shape
  assert d == D
  e = edge_index.shape[1]

  nch = -(-e // (NW * CH))
  nch = ((nch + 31) // 32) * 32
  pad = NW * nch * CH - e
  src = jnp.concatenate(
      [edge_index[0].astype(jnp.int32), jnp.zeros((pad,), jnp.int32)]
  ).reshape(NW, nch // 2, 2 * CH)
  dst = jnp.concatenate(
      [edge_index[1].astype(jnp.int32), jnp.full((pad,), n, jnp.int32)]
  ).reshape(NW, nch // 2, 2 * CH)

  ones2 = jnp.ones((2 * CH, D), jnp.float32)
  z128 = jnp.zeros((CH, D), jnp.float32)

  sc_counts = _make_sc_counts(n, nch)
  sc_agg = _make_sc_agg(n, nch)
  mm, combine, final = _make_tc_kernels(n)

  b_enc = enc_b.reshape(1, D)
  b_l0 = l0_b.reshape(1, D)
  b_l1 = l1_b.reshape(1, D)
  b_dec = dec_b.reshape(1, D)

  cnt = sc_counts(dst, ones2, z128)
  y = mm(x, enc_Wl)
  p = sc_agg(y, src, dst, z128)
  h, y = combine(p, p, cnt, cnt, x, enc_Wr, b_enc, l0_Wl)
  p = sc_agg(y, src, dst, z128)
  h, y = combine(p, p, cnt, cnt, h, l0_Wr, b_l0, l1_Wl)
  p = sc_agg(y, src, dst, z128)
  h, y = combine(p, p, cnt, cnt, h, l1_Wr, b_l1, dec_Wl)
  p = sc_agg(y, src, dst, z128)
  return final(p, p, cnt, cnt, h, dec_Wr, b_dec)


# speedup vs baseline: 1.3921x; 1.0388x over previous
import functools

import jax
import jax.numpy as jnp
from jax import lax
from jax.experimental import pallas as pl
from jax.experimental.pallas import tpu as pltpu
from jax.experimental.pallas import tpu_sc as plsc

D = 128
CH = 64
NB = 3
NC = 2
NS = 16
NW = NC * NS


def _acc_rows(n_nodes):
  u = NS * 8
  return ((n_nodes + 1 + u - 1) // u) * u


def _mesh():
  return plsc.VectorSubcoreMesh(
      core_axis_name="c", subcore_axis_name="s", num_cores=NC, num_subcores=NS
  )


def _make_sc_agg(n_nodes, nch):
  nr = _acc_rows(n_nodes)
  rpt = nr // NS
  SBD = 16
  assert nch % SBD == 0
  GCH = 2 * CH

  out_type = jax.ShapeDtypeStruct((NC, nr, D), jnp.float32)
  scratch = [
      pltpu.VMEM((nch // 2, GCH), jnp.int32),
      pltpu.VMEM((SBD, GCH), jnp.int32),
      pltpu.VMEM((GCH, D), jnp.float32),
      pltpu.VMEM((GCH, D), jnp.float32),
      pltpu.MemorySpace.VMEM_SHARED((nr, D), jnp.float32),
      pltpu.SemaphoreType.DMA,
      pltpu.SemaphoreType.DMA,
      pltpu.SemaphoreType.DMA,
      pltpu.SemaphoreType.DMA,
  ]

  def body(y, srcp, dstp, z128_h, psum,
           src_v, dst_v, rows_a, rows_b, acc_p, sga, sgb, ssa, ssb):
    c = lax.axis_index("c")
    s = lax.axis_index("s")
    w = c * NS + s

    gi = pltpu.async_copy(srcp.at[w], src_v, sgb)
    pltpu.sync_copy(z128_h, rows_a.at[pl.ds(0, CH)])
    zsrc = rows_a.at[pl.ds(0, CH)]
    nz = rpt // CH
    rem = rpt - nz * CH

    @pl.loop(0, nz)
    def _(k):
      pltpu.async_copy(zsrc, acc_p.at[pl.ds(s * rpt + k * CH, CH)], ssa)

    if rem:
      pltpu.async_copy(rows_a.at[pl.ds(0, rem)],
                       acc_p.at[pl.ds(s * rpt + nz * CH, rem)], ssb)

    @pl.loop(0, nz)
    def _(k):
      pltpu.make_async_copy(
          zsrc, acc_p.at[pl.ds(s * rpt + k * CH, CH)], ssa).wait()

    if rem:
      pltpu.make_async_copy(
          rows_a.at[pl.ds(0, rem)],
          acc_p.at[pl.ds(s * rpt + nz * CH, rem)], ssb).wait()
    gi.wait()

    plsc.subcore_barrier()

    nbh = nch // 2 // SBD

    def wait_sb():
      pltpu.make_async_copy(rows_b, acc_p.at[dst_v.at[SBD - 1]], ssb).wait()

    @pl.loop(0, nbh)
    def _(bo):
      @pl.when(bo > 0)
      def _():
        wait_sb()

      pltpu.sync_copy(dstp.at[w, pl.ds(bo * SBD, SBD)], dst_v)

      def pair(j, first):
        q = bo * SBD + j
        ga = pltpu.async_copy(y.at[src_v.at[q]], rows_a, sga)
        if not first:
          pltpu.make_async_copy(rows_b, acc_p.at[dst_v.at[j]], ssb).wait()
        gb = pltpu.async_copy(y.at[src_v.at[q + 1]], rows_b, sgb)
        ga.wait()
        sa = pltpu.async_copy(rows_a, acc_p.at[dst_v.at[j]], ssa, add=True)
        gb.wait()
        pltpu.async_copy(rows_b, acc_p.at[dst_v.at[j + 1]], ssb, add=True)
        sa.wait()

      pair(0, True)

      @pl.loop(1, SBD // 2)
      def _(p):
        pair(2 * p, False)

    wait_sb()

    plsc.subcore_barrier()

    sl = pl.ds(s * rpt, rpt)
    pltpu.sync_copy(acc_p.at[sl], psum.at[c, sl])

  return pl.kernel(body, out_type=out_type, mesh=_mesh(), scratch_types=scratch)


def _make_sc_counts(n_nodes, nch):
  nr = _acc_rows(n_nodes)
  rpt = nr // NS
  nch2 = nch // 2

  out_type = jax.ShapeDtypeStruct((NC, nr, D), jnp.float32)
  scratch = [
      pltpu.VMEM((nch2, 2 * CH), jnp.int32),
      pltpu.VMEM((2 * CH, D), jnp.float32),
      pltpu.VMEM((CH, D), jnp.float32),
      pltpu.MemorySpace.VMEM_SHARED((nr, D), jnp.float32),
      pltpu.SemaphoreType.DMA,
      pltpu.SemaphoreType.DMA,
  ]

  def body(dstp, ones_h, z128_h, cnt,
           dst_v, ones_v, zc_v, acc_c, ssa, ssb):
    c = lax.axis_index("c")
    s = lax.axis_index("s")
    w = c * NS + s

    pltpu.sync_copy(dstp.at[w], dst_v)
    pltpu.sync_copy(z128_h, zc_v)
    pltpu.sync_copy(ones_h, ones_v)

    nz = rpt // CH
    rem = rpt - nz * CH

    @pl.loop(0, nz)
    def _(k):
      pltpu.sync_copy(zc_v, acc_c.at[pl.ds(s * rpt + k * CH, CH)])

    if rem:
      pltpu.sync_copy(zc_v.at[pl.ds(0, rem)],
                      acc_c.at[pl.ds(s * rpt + nz * CH, rem)])

    plsc.subcore_barrier()

    even = nch2 - (nch2 % 2)

    @pl.loop(0, even, step=2)
    def _(m):
      sa = pltpu.async_copy(ones_v, acc_c.at[dst_v.at[m]], ssa, add=True)
      sb = pltpu.async_copy(ones_v, acc_c.at[dst_v.at[m + 1]], ssb, add=True)
      sa.wait()
      sb.wait()

    if nch2 % 2:
      pltpu.async_copy(ones_v, acc_c.at[dst_v.at[nch2 - 1]], ssa,
                       add=True).wait()

    plsc.subcore_barrier()

    sl = pl.ds(s * rpt, rpt)
    pltpu.sync_copy(acc_c.at[sl], cnt.at[c, sl])

  return pl.kernel(body, out_type=out_type, mesh=_mesh(), scratch_types=scratch)


_BR = 1000


def _mm_body(x_ref, w_ref, o_ref):
  o_ref[...] = jnp.dot(
      x_ref[...], w_ref[...],
      preferred_element_type=jnp.float32, precision=lax.Precision.HIGHEST,
  )


def _combine_body(p0, p1, c0, c1, h, wr, b, wl, hn_o, y_o):
  cnt = jnp.maximum(c0[0, :, 0:1] + c1[0, :, 0:1], 1.0)
  z = (p0[0] + p1[0]) / cnt
  z = z + jnp.dot(h[...], wr[...], preferred_element_type=jnp.float32,
                  precision=lax.Precision.HIGHEST)
  z = z + b[...]
  hn = jnp.where(z >= 0, z, 0.1 * z)
  hn_o[...] = hn
  y_o[...] = jnp.dot(hn, wl[...], preferred_element_type=jnp.float32,
                     precision=lax.Precision.HIGHEST)


def _final_body(p0, p1, c0, c1, h, wr, b, o_ref):
  cnt = jnp.maximum(c0[0, :, 0:1] + c1[0, :, 0:1], 1.0)
  z = (p0[0] + p1[0]) / cnt
  z = z + jnp.dot(h[...], wr[...], preferred_element_type=jnp.float32,
                  precision=lax.Precision.HIGHEST)
  z = z + b[...]
  m = jnp.max(z, axis=1, keepdims=True)
  zs = z - m
  o_ref[...] = zs - jnp.log(jnp.sum(jnp.exp(zs), axis=1, keepdims=True))


def _row_spec(width=D):
  return pl.BlockSpec((_BR, width), lambda i: (i, 0))


def _core_spec(core, width=D):
  return pl.BlockSpec((1, _BR, width), lambda i, c=core: (c, i, 0))


def _full_spec(shape):
  return pl.BlockSpec(shape, lambda i: (0,) * len(shape))


def _make_tc_kernels(n):
  grid = (n // _BR,)
  mm = pl.pallas_call(
      _mm_body,
      grid=grid,
      in_specs=[_row_spec(), _full_spec((D, D))],
      out_specs=_row_spec(),
      out_shape=jax.ShapeDtypeStruct((n, D), jnp.float32),
  )
  combine = pl.pallas_call(
      _combine_body,
      grid=grid,
      in_specs=[_core_spec(0), _core_spec(1), _core_spec(0), _core_spec(1),
                _row_spec(), _full_spec((D, D)), _full_spec((1, D)),
                _full_spec((D, D))],
      out_specs=[_row_spec(), _row_spec()],
      out_shape=[jax.ShapeDtypeStruct((n, D), jnp.float32),
                 jax.ShapeDtypeStruct((n, D), jnp.float32)],
  )
  final = pl.pallas_call(
      _final_body,
      grid=grid,
      in_specs=[_core_spec(0), _core_spec(1), _core_spec(0), _core_spec(1),
                _row_spec(), _full_spec((D, D)), _full_spec((1, D))],
      out_specs=_row_spec(),
      out_shape=jax.ShapeDtypeStruct((n, D), jnp.float32),
  )
  return mm, combine, final


def kernel(x, edge_index, enc_Wl, enc_Wr, enc_b, l0_Wl, l0_Wr, l0_b,
           l1_Wl, l1_Wr, l1_b, dec_Wl, dec_Wr, dec_b):
  n, d = x.shape
  assert d == D
  e = edge_index.shape[1]

  nch = -(-e // (NW * CH))
  nch = ((nch + 31) // 32) * 32
  pad = NW * nch * CH - e
  src = jnp.concatenate(
      [edge_index[0].astype(jnp.int32), jnp.zeros((pad,), jnp.int32)]
  ).reshape(NW, nch // 2, 2 * CH)
  dst = jnp.concatenate(
      [edge_index[1].astype(jnp.int32), jnp.full((pad,), n, jnp.int32)]
  ).reshape(NW, nch // 2, 2 * CH)

  ones2 = jnp.ones((2 * CH, D), jnp.float32)
  z128 = jnp.zeros((CH, D), jnp.float32)

  sc_counts = _make_sc_counts(n, nch)
  sc_agg = _make_sc_agg(n, nch)
  mm, combine, final = _make_tc_kernels(n)

  b_enc = enc_b.reshape(1, D)
  b_l0 = l0_b.reshape(1, D)
  b_l1 = l1_b.reshape(1, D)
  b_dec = dec_b.reshape(1, D)

  cnt = sc_counts(dst, ones2, z128)
  y = mm(x, enc_Wl)
  p = sc_agg(y, src, dst, z128)
  h, y = combine(p, p, cnt, cnt, x, enc_Wr, b_enc, l0_Wl)
  p = sc_agg(y, src, dst, z128)
  h, y = combine(p, p, cnt, cnt, h, l0_Wr, b_l0, l1_Wl)
  p = sc_agg(y, src, dst, z128)
  h, y = combine(p, p, cnt, cnt, h, l1_Wr, b_l1, dec_Wl)
  p = sc_agg(y, src, dst, z128)
  return final(p, p, cnt, cnt, h, dec_Wr, b_dec)


# speedup vs baseline: 1.4109x; 1.0135x over previous
import functools

import jax
import jax.numpy as jnp
from jax import lax
from jax.experimental import pallas as pl
from jax.experimental.pallas import tpu as pltpu
from jax.experimental.pallas import tpu_sc as plsc

D = 128
CH = 64
NB = 3
NC = 2
NS = 16
NW = NC * NS


def _acc_rows(n_nodes):
  u = NS * 8
  return ((n_nodes + 1 + u - 1) // u) * u


def _mesh():
  return plsc.VectorSubcoreMesh(
      core_axis_name="c", subcore_axis_name="s", num_cores=NC, num_subcores=NS
  )


def _make_sc_agg(n_nodes, nch):
  nr = _acc_rows(n_nodes)
  rpt = nr // NS
  SBD = 40
  assert (nch // 2) % SBD == 0
  GCH = 2 * CH

  out_type = jax.ShapeDtypeStruct((NC, nr, D), jnp.float32)
  scratch = [
      pltpu.VMEM((nch // 2, GCH), jnp.int32),
      pltpu.VMEM((SBD, GCH), jnp.int32),
      pltpu.VMEM((GCH, D), jnp.float32),
      pltpu.VMEM((GCH, D), jnp.float32),
      pltpu.MemorySpace.VMEM_SHARED((nr, D), jnp.float32),
      pltpu.SemaphoreType.DMA,
      pltpu.SemaphoreType.DMA,
      pltpu.SemaphoreType.DMA,
      pltpu.SemaphoreType.DMA,
  ]

  def body(y, srcp, dstp, z128_h, psum,
           src_v, dst_v, rows_a, rows_b, acc_p, sga, sgb, ssa, ssb):
    c = lax.axis_index("c")
    s = lax.axis_index("s")
    w = c * NS + s

    gi = pltpu.async_copy(srcp.at[w], src_v, sgb)
    pltpu.sync_copy(z128_h, rows_a.at[pl.ds(0, CH)])
    zsrc = rows_a.at[pl.ds(0, CH)]
    nz = rpt // CH
    rem = rpt - nz * CH

    @pl.loop(0, nz)
    def _(k):
      pltpu.async_copy(zsrc, acc_p.at[pl.ds(s * rpt + k * CH, CH)], ssa)

    if rem:
      pltpu.async_copy(rows_a.at[pl.ds(0, rem)],
                       acc_p.at[pl.ds(s * rpt + nz * CH, rem)], ssb)

    @pl.loop(0, nz)
    def _(k):
      pltpu.make_async_copy(
          zsrc, acc_p.at[pl.ds(s * rpt + k * CH, CH)], ssa).wait()

    if rem:
      pltpu.make_async_copy(
          rows_a.at[pl.ds(0, rem)],
          acc_p.at[pl.ds(s * rpt + nz * CH, rem)], ssb).wait()
    gi.wait()

    plsc.subcore_barrier()

    nbh = nch // 2 // SBD

    def wait_sb():
      pltpu.make_async_copy(rows_b, acc_p.at[dst_v.at[SBD - 1]], ssb).wait()

    @pl.loop(0, nbh)
    def _(bo):
      @pl.when(bo > 0)
      def _():
        wait_sb()

      pltpu.sync_copy(dstp.at[w, pl.ds(bo * SBD, SBD)], dst_v)

      def pair(j, first):
        q = bo * SBD + j
        ga = pltpu.async_copy(y.at[src_v.at[q]], rows_a, sga)
        if not first:
          pltpu.make_async_copy(rows_b, acc_p.at[dst_v.at[j]], ssb).wait()
        gb = pltpu.async_copy(y.at[src_v.at[q + 1]], rows_b, sgb)
        ga.wait()
        sa = pltpu.async_copy(rows_a, acc_p.at[dst_v.at[j]], ssa, add=True)
        gb.wait()
        pltpu.async_copy(rows_b, acc_p.at[dst_v.at[j + 1]], ssb, add=True)
        sa.wait()

      pair(0, True)

      @pl.loop(1, SBD // 2)
      def _(p):
        pair(2 * p, False)

    wait_sb()

    plsc.subcore_barrier()

    sl = pl.ds(s * rpt, rpt)
    pltpu.sync_copy(acc_p.at[sl], psum.at[c, sl])

  return pl.kernel(body, out_type=out_type, mesh=_mesh(), scratch_types=scratch)


def _make_sc_counts(n_nodes, nch):
  nr = _acc_rows(n_nodes)
  rpt = nr // NS
  nch2 = nch // 2

  out_type = jax.ShapeDtypeStruct((NC, nr, D), jnp.float32)
  scratch = [
      pltpu.VMEM((nch2, 2 * CH), jnp.int32),
      pltpu.VMEM((2 * CH, D), jnp.float32),
      pltpu.VMEM((CH, D), jnp.float32),
      pltpu.MemorySpace.VMEM_SHARED((nr, D), jnp.float32),
      pltpu.SemaphoreType.DMA,
      pltpu.SemaphoreType.DMA,
  ]

  def body(dstp, ones_h, z128_h, cnt,
           dst_v, ones_v, zc_v, acc_c, ssa, ssb):
    c = lax.axis_index("c")
    s = lax.axis_index("s")
    w = c * NS + s

    pltpu.sync_copy(dstp.at[w], dst_v)
    pltpu.sync_copy(z128_h, zc_v)
    pltpu.sync_copy(ones_h, ones_v)

    nz = rpt // CH
    rem = rpt - nz * CH

    @pl.loop(0, nz)
    def _(k):
      pltpu.sync_copy(zc_v, acc_c.at[pl.ds(s * rpt + k * CH, CH)])

    if rem:
      pltpu.sync_copy(zc_v.at[pl.ds(0, rem)],
                      acc_c.at[pl.ds(s * rpt + nz * CH, rem)])

    plsc.subcore_barrier()

    even = nch2 - (nch2 % 2)

    @pl.loop(0, even, step=2)
    def _(m):
      sa = pltpu.async_copy(ones_v, acc_c.at[dst_v.at[m]], ssa, add=True)
      sb = pltpu.async_copy(ones_v, acc_c.at[dst_v.at[m + 1]], ssb, add=True)
      sa.wait()
      sb.wait()

    if nch2 % 2:
      pltpu.async_copy(ones_v, acc_c.at[dst_v.at[nch2 - 1]], ssa,
                       add=True).wait()

    plsc.subcore_barrier()

    sl = pl.ds(s * rpt, rpt)
    pltpu.sync_copy(acc_c.at[sl], cnt.at[c, sl])

  return pl.kernel(body, out_type=out_type, mesh=_mesh(), scratch_types=scratch)


_BR = 1000


def _mm_body(x_ref, w_ref, o_ref):
  o_ref[...] = jnp.dot(
      x_ref[...], w_ref[...],
      preferred_element_type=jnp.float32, precision=lax.Precision.HIGHEST,
  )


def _combine_body(p0, p1, c0, c1, h, wr, b, wl, hn_o, y_o):
  cnt = jnp.maximum(c0[0, :, 0:1] + c1[0, :, 0:1], 1.0)
  z = (p0[0] + p1[0]) / cnt
  z = z + jnp.dot(h[...], wr[...], preferred_element_type=jnp.float32,
                  precision=lax.Precision.HIGHEST)
  z = z + b[...]
  hn = jnp.where(z >= 0, z, 0.1 * z)
  hn_o[...] = hn
  y_o[...] = jnp.dot(hn, wl[...], preferred_element_type=jnp.float32,
                     precision=lax.Precision.HIGHEST)


def _final_body(p0, p1, c0, c1, h, wr, b, o_ref):
  cnt = jnp.maximum(c0[0, :, 0:1] + c1[0, :, 0:1], 1.0)
  z = (p0[0] + p1[0]) / cnt
  z = z + jnp.dot(h[...], wr[...], preferred_element_type=jnp.float32,
                  precision=lax.Precision.HIGHEST)
  z = z + b[...]
  m = jnp.max(z, axis=1, keepdims=True)
  zs = z - m
  o_ref[...] = zs - jnp.log(jnp.sum(jnp.exp(zs), axis=1, keepdims=True))


def _row_spec(width=D):
  return pl.BlockSpec((_BR, width), lambda i: (i, 0))


def _core_spec(core, width=D):
  return pl.BlockSpec((1, _BR, width), lambda i, c=core: (c, i, 0))


def _full_spec(shape):
  return pl.BlockSpec(shape, lambda i: (0,) * len(shape))


def _make_tc_kernels(n):
  grid = (n // _BR,)
  mm = pl.pallas_call(
      _mm_body,
      grid=grid,
      in_specs=[_row_spec(), _full_spec((D, D))],
      out_specs=_row_spec(),
      out_shape=jax.ShapeDtypeStruct((n, D), jnp.float32),
  )
  combine = pl.pallas_call(
      _combine_body,
      grid=grid,
      in_specs=[_core_spec(0), _core_spec(1), _core_spec(0), _core_spec(1),
                _row_spec(), _full_spec((D, D)), _full_spec((1, D)),
                _full_spec((D, D))],
      out_specs=[_row_spec(), _row_spec()],
      out_shape=[jax.ShapeDtypeStruct((n, D), jnp.float32),
                 jax.ShapeDtypeStruct((n, D), jnp.float32)],
  )
  final = pl.pallas_call(
      _final_body,
      grid=grid,
      in_specs=[_core_spec(0), _core_spec(1), _core_spec(0), _core_spec(1),
                _row_spec(), _full_spec((D, D)), _full_spec((1, D))],
      out_specs=_row_spec(),
      out_shape=jax.ShapeDtypeStruct((n, D), jnp.float32),
  )
  return mm, combine, final


def kernel(x, edge_index, enc_Wl, enc_Wr, enc_b, l0_Wl, l0_Wr, l0_b,
           l1_Wl, l1_Wr, l1_b, dec_Wl, dec_Wr, dec_b):
  n, d = x.shape
  assert d == D
  e = edge_index.shape[1]

  nch = -(-e // (NW * CH))
  nch = ((nch + 31) // 32) * 32
  pad = NW * nch * CH - e
  src = jnp.concatenate(
      [edge_index[0].astype(jnp.int32), jnp.zeros((pad,), jnp.int32)]
  ).reshape(NW, nch // 2, 2 * CH)
  dst = jnp.concatenate(
      [edge_index[1].astype(jnp.int32), jnp.full((pad,), n, jnp.int32)]
  ).reshape(NW, nch // 2, 2 * CH)

  ones2 = jnp.ones((2 * CH, D), jnp.float32)
  z128 = jnp.zeros((CH, D), jnp.float32)

  sc_counts = _make_sc_counts(n, nch)
  sc_agg = _make_sc_agg(n, nch)
  mm, combine, final = _make_tc_kernels(n)

  b_enc = enc_b.reshape(1, D)
  b_l0 = l0_b.reshape(1, D)
  b_l1 = l1_b.reshape(1, D)
  b_dec = dec_b.reshape(1, D)

  cnt = sc_counts(dst, ones2, z128)
  y = mm(x, enc_Wl)
  p = sc_agg(y, src, dst, z128)
  h, y = combine(p, p, cnt, cnt, x, enc_Wr, b_enc, l0_Wl)
  p = sc_agg(y, src, dst, z128)
  h, y = combine(p, p, cnt, cnt, h, l0_Wr, b_l0, l1_Wl)
  p = sc_agg(y, src, dst, z128)
  h, y = combine(p, p, cnt, cnt, h, l1_Wr, b_l1, dec_Wl)
  p = sc_agg(y, src, dst, z128)
  return final(p, p, cnt, cnt, h, dec_Wr, b_dec)
